# Initial kernel scaffold; baseline (speedup 1.0000x reference)
import jax
import jax.numpy as jnp
from jax.experimental import pallas as pl


def kernel(x, table):
    raise NotImplementedError("write your pallas kernel here")


# speedup vs baseline: 1.6879x; 1.6879x over previous
import functools
import math

import jax
import jax.numpy as jnp
import numpy as np
from jax import lax
from jax.experimental import pallas as pl
from jax.experimental.pallas import tpu as pltpu
from jax.experimental.pallas import tpu_sc as plsc

_D = 128
_SEQ = 200
_BATCH = 1024
_SCALE = math.sqrt(float(_D))

_NC = 2
_NS = 16
_NW = _NC * _NS

_B = _BATCH * _SEQ
_BPW = _B // _NW
_CH = 40
_NCHUNK = _BPW // _CH
_NPHASE = _SEQ // _CH


def _make_pe():
    position = np.arange(_SEQ, dtype=np.float32)[:, None]
    div_term = np.exp(
        np.arange(0, _D, 2, dtype=np.float32) * (-math.log(10000.0) / _D))
    pe = np.zeros((_SEQ, _D), dtype=np.float32)
    pe[:, 0::2] = np.sin(position * div_term)
    pe[:, 1::2] = np.cos(position * div_term)
    return pe


_PE = _make_pe()


def _sc_body(x_hbm, pe_hbm, table_hbm, out_hbm,
             idx_v, pe_v, rows0, rows1, res0, res1,
             sem_g0, sem_g1, sem_s0, sem_s1):
    wid = lax.axis_index("s") * _NC + lax.axis_index("c")
    base = wid * _BPW

    pltpu.sync_copy(x_hbm.at[pl.ds(base, _BPW)], idx_v)
    pltpu.sync_copy(pe_hbm, pe_v)

    rows = (rows0, rows1)
    res = (res0, res1)
    sem_g = (sem_g0, sem_g1)
    sem_s = (sem_s0, sem_s1)

    def gather_desc(c, b):
        off = pl.multiple_of(c * _CH, 8)
        return pltpu.make_async_copy(
            table_hbm.at[idx_v.at[pl.ds(off, _CH)]], rows[b], sem_g[b])

    def scatter_desc(c, b):
        off = pl.multiple_of(base + c * _CH, 8)
        return pltpu.make_async_copy(
            res[b], out_hbm.at[pl.ds(off, _CH)], sem_s[b])

    gather_desc(0, 0).start()
    gather_desc(1, 1).start()

    def step(i, carry):
        c0 = i * 2
        for b in range(2):
            c = c0 + b
            gather_desc(c, b).wait()

            @pl.when(c >= 2)
            def _wait_prev_scatter():
                scatter_desc(c - 2, b).wait()

            p0 = lax.rem(c, _NPHASE) * _CH
            for t in range(_CH):
                p = p0 + t
                for j in range(_D // 16):
                    sl = pl.ds(j * 16, 16)
                    res[b][t, sl] = rows[b][t, sl] * _SCALE + pe_v[p, sl]

            @pl.when(c + 2 < _NCHUNK)
            def _prefetch():
                gather_desc(c + 2, b).start()

            scatter_desc(c, b).start()
        return carry

    lax.fori_loop(0, _NCHUNK // 2, step, None)
    scatter_desc(_NCHUNK - 2, 0).wait()
    scatter_desc(_NCHUNK - 1, 1).wait()


_sc_call = functools.partial(
    pl.kernel,
    mesh=plsc.VectorSubcoreMesh(core_axis_name="c", subcore_axis_name="s"),
    out_type=jax.ShapeDtypeStruct((_B, _D), jnp.float32),
    scratch_types=[
        pltpu.VMEM((_BPW,), jnp.int32),
        pltpu.VMEM((_SEQ, _D), jnp.float32),
        pltpu.VMEM((_CH, _D), jnp.float32),
        pltpu.VMEM((_CH, _D), jnp.float32),
        pltpu.VMEM((_CH, _D), jnp.float32),
        pltpu.VMEM((_CH, _D), jnp.float32),
        pltpu.SemaphoreType.DMA,
        pltpu.SemaphoreType.DMA,
        pltpu.SemaphoreType.DMA,
        pltpu.SemaphoreType.DMA,
    ],
)(_sc_body)


def kernel(x, table):
    xf = jnp.asarray(x).reshape(_B).astype(jnp.int32)
    pe = jnp.asarray(_PE)
    out = _sc_call(xf, pe, table)
    return out.reshape(_BATCH, _SEQ, _D)
